# baseline (speedup 1.0000x reference)
import functools

import jax
import jax.numpy as jnp
from jax import lax
from jax.experimental import pallas as pl
from jax.experimental.pallas import tpu as pltpu
from jax.experimental.pallas import tpu_sc as plsc

NUM_CORES = 2
NUM_SUBCORES = 16
NUM_WORKERS = NUM_CORES * NUM_SUBCORES

D = 128
CHUNK = 128
NBUF = 4


def _gather_body(n_chunks, x_hbm, w_hbm, out_hbm, idx_v, bufs, gsems, wsems):
  wid = lax.axis_index("s") * NUM_CORES + lax.axis_index("c")
  base = wid * n_chunks

  pltpu.sync_copy(x_hbm.at[pl.ds(base, n_chunks)], idx_v)

  def start_gather(c, b):
    pltpu.async_copy(w_hbm.at[idx_v.at[c]], bufs[b], gsems[b])

  def wait_gather(b):
    pltpu.make_async_copy(w_hbm.at[idx_v.at[0]], bufs[b], gsems[b]).wait()

  def start_write(c, b):
    pltpu.async_copy(bufs[b], out_hbm.at[pl.ds((base + c) * CHUNK, CHUNK)],
                     wsems[b])

  def wait_write(b):
    pltpu.make_async_copy(bufs[b], out_hbm.at[pl.ds(0, CHUNK)], wsems[b]).wait()

  for b in range(NBUF):
    start_gather(b, b)

  n_groups = n_chunks // NBUF

  def loop_body(j, carry):
    for b in range(NBUF):
      c = NBUF * j + b
      wait_gather(b)
      start_write(c, b)
      wait_write(b)
      start_gather(c + NBUF, b)
    return carry

  lax.fori_loop(0, n_groups - 1, loop_body, 0, unroll=False)

  for b in range(NBUF):
    c = NBUF * (n_groups - 1) + b
    wait_gather(b)
    start_write(c, b)
  for b in range(NBUF):
    wait_write(b)


def kernel(x, weight):
  N, S = x.shape
  B = N * S
  assert B % (NUM_WORKERS * CHUNK) == 0
  b_per_w = B // NUM_WORKERS
  n_chunks = b_per_w // CHUNK
  assert n_chunks % NBUF == 0

  flat_x = jnp.transpose(x, (1, 0)).reshape(B // CHUNK, CHUNK)

  mesh = plsc.VectorSubcoreMesh(
      core_axis_name="c", subcore_axis_name="s",
      num_cores=NUM_CORES, num_subcores=NUM_SUBCORES)

  grid_kernel = pl.kernel(
      functools.partial(_gather_body, n_chunks),
      out_type=jax.ShapeDtypeStruct((B, D), jnp.float32),
      mesh=mesh,
      scratch_types=[
          pltpu.VMEM((n_chunks, CHUNK), jnp.int32),
          [pltpu.VMEM((CHUNK, D), jnp.float32) for _ in range(NBUF)],
          [pltpu.SemaphoreType.DMA for _ in range(NBUF)],
          [pltpu.SemaphoreType.DMA for _ in range(NBUF)],
      ],
  )
  out = grid_kernel(flat_x, weight)
  return out.reshape(S, N, D).transpose(1, 0, 2)
